# Initial kernel scaffold; baseline (speedup 1.0000x reference)
import jax
import jax.numpy as jnp
from jax.experimental import pallas as pl


def kernel(x, edge_index, eps):
    raise NotImplementedError("write your pallas kernel here")


# speedup vs baseline: 7.6226x; 7.6226x over previous
import jax
import jax.numpy as jnp
from jax import lax
from jax.experimental import pallas as pl
from jax.experimental.pallas import tpu as pltpu
from jax.experimental.pallas import tpu_sc as plsc

N_NODES = 10000
D_FEAT = 128
N_EDGES = 320000

NC = 2
NS = 16
NW = NC * NS
E_PER_W = N_EDGES // NW
CHUNK = 80
N_CHUNKS = E_PER_W // CHUNK
N_PAD = 10240
ROWS_PER_TILE = N_PAD // NS


def _sc_partials_kernel(x_hbm, src_hbm, dst_hbm, zeros_hbm, out_hbm,
                        src_idx, dst_idx, rows, acc, sem):
  cid = lax.axis_index("c")
  sid = lax.axis_index("s")
  wid = sid * NC + cid

  pltpu.sync_copy(zeros_hbm, acc.at[pl.ds(sid * ROWS_PER_TILE, ROWS_PER_TILE)])
  pltpu.sync_copy(src_hbm.at[wid], src_idx)
  pltpu.sync_copy(dst_hbm.at[wid], dst_idx)
  plsc.subcore_barrier()

  def chunk_step(j, carry):
    pltpu.async_copy(x_hbm.at[src_idx.at[j]], rows, sem).wait()
    pltpu.sync_copy(rows, acc.at[dst_idx.at[j]], add=True)
    return carry

  lax.fori_loop(0, N_CHUNKS, chunk_step, 0)
  plsc.subcore_barrier()
  pltpu.sync_copy(acc.at[pl.ds(sid * ROWS_PER_TILE, ROWS_PER_TILE)],
                  out_hbm.at[cid, pl.ds(sid * ROWS_PER_TILE, ROWS_PER_TILE)])


def _combine_kernel(eps_ref, x_ref, p_ref, o_ref):
  scale = 1.0 + eps_ref[0]
  o_ref[...] = x_ref[...] * scale + p_ref[0] + p_ref[1]


@jax.jit
def kernel(x, edge_index, eps):
  src = edge_index[0].astype(jnp.int32).reshape(NW, N_CHUNKS, CHUNK)
  dst = edge_index[1].astype(jnp.int32).reshape(NW, N_CHUNKS, CHUNK)
  zeros = jnp.zeros((ROWS_PER_TILE, D_FEAT), dtype=jnp.float32)

  mesh = plsc.VectorSubcoreMesh(core_axis_name="c", subcore_axis_name="s")
  partials = pl.kernel(
      _sc_partials_kernel,
      out_type=jax.ShapeDtypeStruct((NC, N_PAD, D_FEAT), jnp.float32),
      mesh=mesh,
      scratch_types=[
          pltpu.VMEM((N_CHUNKS, CHUNK), jnp.int32),
          pltpu.VMEM((N_CHUNKS, CHUNK), jnp.int32),
          pltpu.VMEM((CHUNK, D_FEAT), jnp.float32),
          pltpu.VMEM_SHARED((N_PAD, D_FEAT), jnp.float32),
          pltpu.SemaphoreType.DMA,
      ],
  )(x, src, dst, zeros)

  rows_blk = 1000
  grid = N_NODES // rows_blk
  out = pl.pallas_call(
      _combine_kernel,
      out_shape=jax.ShapeDtypeStruct((N_NODES, D_FEAT), jnp.float32),
      grid=(grid,),
      in_specs=[
          pl.BlockSpec(memory_space=pltpu.SMEM),
          pl.BlockSpec((rows_blk, D_FEAT), lambda i: (i, 0)),
          pl.BlockSpec((NC, rows_blk, D_FEAT), lambda i: (0, i, 0)),
      ],
      out_specs=pl.BlockSpec((rows_blk, D_FEAT), lambda i: (i, 0)),
  )(eps, x, partials)
  return out
